# Initial kernel scaffold; baseline (speedup 1.0000x reference)
import jax
import jax.numpy as jnp
from jax.experimental import pallas as pl


def kernel(pred_scores, pred_dist, pred_bboxes, anchors, strides, gt_labels, gt_bboxes, mask_gt):
    raise NotImplementedError("write your pallas kernel here")


# speedup vs baseline: 22.3425x; 22.3425x over previous
import jax
import jax.numpy as jnp
import numpy as np
from jax.experimental import pallas as pl

REG_MAX = 16
NC = 80
TOPK = 10
BOX_W, CLS_W, DFL_W = 7.5, 0.5, 1.5
EPS = 1e-7


def _atan_pos(z):
    inv = z > 1.0
    x = jnp.where(inv, 1.0 / z, z)
    x2 = x * x
    p = jnp.float32(-0.0117212)
    p = p * x2 + jnp.float32(0.05265332)
    p = p * x2 + jnp.float32(-0.11643287)
    p = p * x2 + jnp.float32(0.19354346)
    p = p * x2 + jnp.float32(-0.33262347)
    p = p * x2 + jnp.float32(0.99997726)
    r = x * p
    return jnp.where(inv, jnp.float32(np.pi / 2) - r, r)


def _loss_kernel(scores_ref, dist_ref, pboxT_ref, anchT_ref, strideT_ref,
                 gtb_ref, gtlab_ref, mg_ref, out_ref):
    A = scores_ref.shape[1]
    C = scores_ref.shape[2]
    G = gtb_ref.shape[1]
    D = 4 * REG_MAX

    s = scores_ref[0]
    softplus_sum = jnp.sum(jnp.maximum(s, 0.0) + jnp.log1p(jnp.exp(-jnp.abs(s))))
    smax_col = jnp.max(s, axis=1, keepdims=True)
    smax = jax.nn.sigmoid(jnp.transpose(smax_col))

    pbT = pboxT_ref[0]
    px1 = pbT[0:1, :]
    py1 = pbT[1:2, :]
    px2 = pbT[2:3, :]
    py2 = pbT[3:4, :]
    gtb = gtb_ref[0]
    gx1 = gtb[:, 0:1]
    gy1 = gtb[:, 1:2]
    gx2 = gtb[:, 2:3]
    gy2 = gtb[:, 3:4]

    iw = jnp.clip(jnp.minimum(px2, gx2) - jnp.maximum(px1, gx1), 0.0, None)
    ih = jnp.clip(jnp.minimum(py2, gy2) - jnp.maximum(py1, gy1), 0.0, None)
    inter = iw * ih
    area_p = (px2 - px1) * (py2 - py1)
    area_g = (gx2 - gx1) * (gy2 - gy1)
    iou = inter / (area_p + area_g - inter + EPS)

    mg = mg_ref[0]
    i2 = iou * iou
    i6 = i2 * i2 * i2
    align = jnp.sqrt(smax) * i6 * mg

    iota_a = jax.lax.broadcasted_iota(jnp.int32, (G, A), 1)
    work = align
    mask_pos = jnp.zeros((G, A), dtype=jnp.float32)
    for _ in range(TOPK):
        v = jnp.max(work, axis=1, keepdims=True)
        idx = jnp.min(jnp.where(work == v, iota_a, A), axis=1, keepdims=True)
        sel = (iota_a == idx)
        mask_pos = jnp.where(sel & (v > 0.0), 1.0, mask_pos)
        work = jnp.where(sel, -1.0, work)

    masked_iou = iou * mask_pos
    iou_sc = jnp.max(masked_iou, axis=0, keepdims=True)
    fgm = (iou_sc > 0.0).astype(jnp.float32)
    iota_g = jax.lax.broadcasted_iota(jnp.int32, (G, A), 0)
    gidx = jnp.min(jnp.where(masked_iou == iou_sc, iota_g, G), axis=0,
                   keepdims=True)
    onehot_g = (iota_g == gidx).astype(jnp.float32)

    tbx1 = jnp.sum(onehot_g * gx1, axis=0, keepdims=True)
    tby1 = jnp.sum(onehot_g * gy1, axis=0, keepdims=True)
    tbx2 = jnp.sum(onehot_g * gx2, axis=0, keepdims=True)
    tby2 = jnp.sum(onehot_g * gy2, axis=0, keepdims=True)

    lab = jnp.clip(gtlab_ref[0], 0.0, C - 1)
    iota_c = jax.lax.broadcasted_iota(jnp.int32, (G, C), 1)
    onehot_lab = (iota_c == lab.astype(jnp.int32)).astype(jnp.float32)
    wg = onehot_g * iou_sc
    m_gc = jnp.dot(wg, s, preferred_element_type=jnp.float32)
    bce_g = jnp.sum(m_gc * onehot_lab)
    score_sum = jnp.sum(iou_sc)

    ciw = jnp.clip(jnp.minimum(px2, tbx2) - jnp.maximum(px1, tbx1), 0.0, None)
    cih = jnp.clip(jnp.minimum(py2, tby2) - jnp.maximum(py1, tby1), 0.0, None)
    c_inter = ciw * cih
    w1 = jnp.clip(px2 - px1, EPS, None)
    h1 = jnp.clip(py2 - py1, EPS, None)
    w2 = jnp.clip(tbx2 - tbx1, EPS, None)
    h2 = jnp.clip(tby2 - tby1, EPS, None)
    c_union = w1 * h1 + w2 * h2 - c_inter + EPS
    c_iou = c_inter / c_union
    cw = jnp.maximum(px2, tbx2) - jnp.minimum(px1, tbx1)
    ch = jnp.maximum(py2, tby2) - jnp.minimum(py1, tby1)
    c2 = cw * cw + ch * ch + EPS
    rho2 = ((px1 + px2 - tbx1 - tbx2) * 0.5) ** 2 + ((py1 + py2 - tby1 - tby2) * 0.5) ** 2
    v_ar = (4.0 / np.pi ** 2) * (_atan_pos(w2 / h2) - _atan_pos(w1 / h1)) ** 2
    alpha = v_ar / (1.0 - c_iou + v_ar + EPS)
    ciou = jnp.clip(c_iou - (rho2 / c2 + v_ar * alpha), -1.0, 1.0)
    box_sum = jnp.sum((1.0 - ciou) * fgm)
    nfg = jnp.sum(fgm)

    ax = anchT_ref[0:1, :]
    ay = anchT_ref[1:2, :]
    st = strideT_ref[0:1, :]
    tds = (jnp.clip((ax - tbx1) / st, 0.0, REG_MAX - 1.01),
           jnp.clip((ay - tby1) / st, 0.0, REG_MAX - 1.01),
           jnp.clip((tbx2 - ax) / st, 0.0, REG_MAX - 1.01),
           jnp.clip((tby2 - ay) / st, 0.0, REG_MAX - 1.01))
    iota_d = jax.lax.broadcasted_iota(jnp.int32, (D, A), 0)
    v_w = jnp.zeros((D, A), dtype=jnp.float32)
    for i in range(4):
        td = tds[i]
        tl = jnp.clip(jnp.floor(td), 0.0, REG_MAX - 1)
        tli = tl.astype(jnp.int32)
        tri = jnp.minimum(tli + 1, REG_MAX - 1)
        wr = jnp.clip(td - tl, 0.0, 1.0)
        wl = 1.0 - wr
        eq_l = (iota_d == i * REG_MAX + tli).astype(jnp.float32)
        eq_r = (iota_d == i * REG_MAX + tri).astype(jnp.float32)
        v_w = v_w + fgm * (wl * eq_l + wr * eq_r)

    dist = dist_ref[0]
    gterm = jnp.dot(v_w, dist, preferred_element_type=jnp.float32)
    eye_d = (jax.lax.broadcasted_iota(jnp.int32, (D, D), 0)
             == jax.lax.broadcasted_iota(jnp.int32, (D, D), 1)).astype(jnp.float32)
    gath = jnp.sum(gterm * eye_d)

    dmax = jnp.max(dist, axis=1, keepdims=True)
    edist = jnp.exp(dist - dmax)
    gsel = (jax.lax.broadcasted_iota(jnp.int32, (D, 4), 0) // REG_MAX
            == jax.lax.broadcasted_iota(jnp.int32, (D, 4), 1)).astype(jnp.float32)
    gsum = jnp.dot(edist, gsel, preferred_element_type=jnp.float32)
    lse = dmax + jnp.log(gsum)
    fglse = jnp.dot(fgm, lse, preferred_element_type=jnp.float32)
    dfl_sum = jnp.sum(fglse) - gath

    zero = jnp.zeros((), jnp.float32)
    row = jnp.concatenate(
        [p.reshape(1, 1, 1) for p in
         (softplus_sum, bce_g, score_sum, box_sum, nfg, dfl_sum, zero, zero)],
        axis=2)
    out_ref[...] = row


@jax.jit
def kernel(pred_scores, pred_dist, pred_bboxes, anchors, strides,
           gt_labels, gt_bboxes, mask_gt):
    B, A, C = pred_scores.shape
    G = gt_bboxes.shape[1]
    pboxT = jnp.transpose(pred_bboxes, (0, 2, 1))
    anchT = jnp.transpose(anchors, (1, 0))
    strideT = strides.reshape(1, A)
    gtlab = gt_labels.astype(jnp.float32)[:, :, None]
    mg = mask_gt.astype(jnp.float32)[:, :, None]

    partials = pl.pallas_call(
        _loss_kernel,
        grid=(B,),
        in_specs=[
            pl.BlockSpec((1, A, C), lambda b: (b, 0, 0)),
            pl.BlockSpec((1, A, 4 * REG_MAX), lambda b: (b, 0, 0)),
            pl.BlockSpec((1, 4, A), lambda b: (b, 0, 0)),
            pl.BlockSpec((2, A), lambda b: (0, 0)),
            pl.BlockSpec((1, A), lambda b: (0, 0)),
            pl.BlockSpec((1, G, 4), lambda b: (b, 0, 0)),
            pl.BlockSpec((1, G, 1), lambda b: (b, 0, 0)),
            pl.BlockSpec((1, G, 1), lambda b: (b, 0, 0)),
        ],
        out_specs=pl.BlockSpec((1, 1, 8), lambda b: (b, 0, 0)),
        out_shape=jax.ShapeDtypeStruct((B, 1, 8), jnp.float32),
    )(pred_scores, pred_dist, pboxT, anchT, strideT, gt_bboxes, gtlab, mg)

    partials = partials[:, 0, :]
    softplus_sum = jnp.sum(partials[:, 0])
    bce_g = jnp.sum(partials[:, 1])
    score_sum = jnp.maximum(jnp.sum(partials[:, 2]), 1.0)
    box_sum = jnp.sum(partials[:, 3])
    nfg = jnp.sum(partials[:, 4])
    dfl_sum = jnp.sum(partials[:, 5])

    loss_cls = (softplus_sum - bce_g) / score_sum
    loss_box = box_sum / nfg
    loss_dfl = dfl_sum / nfg / 4.0
    return BOX_W * loss_box + CLS_W * loss_cls + DFL_W * loss_dfl


# speedup vs baseline: 25.9031x; 1.1594x over previous
import jax
import jax.numpy as jnp
import numpy as np
from jax.experimental import pallas as pl
from jax.experimental.pallas import tpu as pltpu

REG_MAX = 16
NC = 80
TOPK = 10
BOX_W, CLS_W, DFL_W = 7.5, 0.5, 1.5
EPS = 1e-7


def _atan_pos(z):
    inv = z > 1.0
    x = jnp.where(inv, 1.0 / z, z)
    x2 = x * x
    p = jnp.float32(-0.0117212)
    p = p * x2 + jnp.float32(0.05265332)
    p = p * x2 + jnp.float32(-0.11643287)
    p = p * x2 + jnp.float32(0.19354346)
    p = p * x2 + jnp.float32(-0.33262347)
    p = p * x2 + jnp.float32(0.99997726)
    r = x * p
    return jnp.where(inv, jnp.float32(np.pi / 2) - r, r)


def _loss_kernel(scores_ref, dist_ref, pboxT_ref, anchT_ref, strideT_ref,
                 gtb_ref, gtlab_ref, mg_ref, out_ref):
    A = scores_ref.shape[1]
    C = scores_ref.shape[2]
    G = gtb_ref.shape[1]
    D = 4 * REG_MAX

    s = scores_ref[0]
    softplus_sum = jnp.sum(jnp.maximum(s, 0.0) + jnp.log1p(jnp.exp(-jnp.abs(s))))
    smax_col = jnp.max(s, axis=1, keepdims=True)
    smax = jax.nn.sigmoid(jnp.transpose(smax_col))

    pbT = pboxT_ref[0]
    px1 = pbT[0:1, :]
    py1 = pbT[1:2, :]
    px2 = pbT[2:3, :]
    py2 = pbT[3:4, :]
    gtb = gtb_ref[0]
    gx1 = gtb[:, 0:1]
    gy1 = gtb[:, 1:2]
    gx2 = gtb[:, 2:3]
    gy2 = gtb[:, 3:4]

    iw = jnp.clip(jnp.minimum(px2, gx2) - jnp.maximum(px1, gx1), 0.0, None)
    ih = jnp.clip(jnp.minimum(py2, gy2) - jnp.maximum(py1, gy1), 0.0, None)
    inter = iw * ih
    area_p = (px2 - px1) * (py2 - py1)
    area_g = (gx2 - gx1) * (gy2 - gy1)
    iou = inter / (area_p + area_g - inter + EPS)

    mg = mg_ref[0]
    i2 = iou * iou
    i6 = i2 * i2 * i2
    align = jnp.sqrt(smax) * i6 * mg

    iota_a = jax.lax.broadcasted_iota(jnp.int32, (G, A), 1)
    work = align
    for _ in range(TOPK):
        v = jnp.max(work, axis=1, keepdims=True)
        idx = jnp.min(jnp.where(work == v, iota_a, A), axis=1, keepdims=True)
        work = jnp.where(iota_a == idx, jnp.float32(-1.0), work)
    mask_pos = ((work < 0.0) & (align > 0.0)).astype(jnp.float32)

    masked_iou = iou * mask_pos
    iou_sc = jnp.max(masked_iou, axis=0, keepdims=True)
    fgm = (iou_sc > 0.0).astype(jnp.float32)
    iota_g = jax.lax.broadcasted_iota(jnp.int32, (G, A), 0)
    gidx = jnp.min(jnp.where(masked_iou == iou_sc, iota_g, G), axis=0,
                   keepdims=True)
    onehot_g = (iota_g == gidx).astype(jnp.float32)

    tbx1 = jnp.sum(onehot_g * gx1, axis=0, keepdims=True)
    tby1 = jnp.sum(onehot_g * gy1, axis=0, keepdims=True)
    tbx2 = jnp.sum(onehot_g * gx2, axis=0, keepdims=True)
    tby2 = jnp.sum(onehot_g * gy2, axis=0, keepdims=True)

    lab = jnp.clip(gtlab_ref[0], 0.0, C - 1)
    iota_c = jax.lax.broadcasted_iota(jnp.int32, (G, C), 1)
    onehot_lab = (iota_c == lab.astype(jnp.int32)).astype(jnp.float32)
    wg = onehot_g * iou_sc
    m_gc = jnp.dot(wg, s, preferred_element_type=jnp.float32)
    bce_g = jnp.sum(m_gc * onehot_lab)
    score_sum = jnp.sum(iou_sc)

    ciw = jnp.clip(jnp.minimum(px2, tbx2) - jnp.maximum(px1, tbx1), 0.0, None)
    cih = jnp.clip(jnp.minimum(py2, tby2) - jnp.maximum(py1, tby1), 0.0, None)
    c_inter = ciw * cih
    w1 = jnp.clip(px2 - px1, EPS, None)
    h1 = jnp.clip(py2 - py1, EPS, None)
    w2 = jnp.clip(tbx2 - tbx1, EPS, None)
    h2 = jnp.clip(tby2 - tby1, EPS, None)
    c_union = w1 * h1 + w2 * h2 - c_inter + EPS
    c_iou = c_inter / c_union
    cw = jnp.maximum(px2, tbx2) - jnp.minimum(px1, tbx1)
    ch = jnp.maximum(py2, tby2) - jnp.minimum(py1, tby1)
    c2 = cw * cw + ch * ch + EPS
    rho2 = ((px1 + px2 - tbx1 - tbx2) * 0.5) ** 2 + ((py1 + py2 - tby1 - tby2) * 0.5) ** 2
    v_ar = (4.0 / np.pi ** 2) * (_atan_pos(w2 / h2) - _atan_pos(w1 / h1)) ** 2
    alpha = v_ar / (1.0 - c_iou + v_ar + EPS)
    ciou = jnp.clip(c_iou - (rho2 / c2 + v_ar * alpha), -1.0, 1.0)
    box_sum = jnp.sum((1.0 - ciou) * fgm)
    nfg = jnp.sum(fgm)

    ax = anchT_ref[0:1, :]
    ay = anchT_ref[1:2, :]
    st = strideT_ref[0:1, :]
    tds = (jnp.clip((ax - tbx1) / st, 0.0, REG_MAX - 1.01),
           jnp.clip((ay - tby1) / st, 0.0, REG_MAX - 1.01),
           jnp.clip((tbx2 - ax) / st, 0.0, REG_MAX - 1.01),
           jnp.clip((tby2 - ay) / st, 0.0, REG_MAX - 1.01))
    iota_r = jax.lax.broadcasted_iota(jnp.int32, (REG_MAX, A), 0)
    v_blocks = []
    for i in range(4):
        td = tds[i]
        tl = jnp.clip(jnp.floor(td), 0.0, REG_MAX - 1)
        tli = tl.astype(jnp.int32)
        tri = jnp.minimum(tli + 1, REG_MAX - 1)
        wr = jnp.clip(td - tl, 0.0, 1.0)
        wl = 1.0 - wr
        eq_l = (iota_r == tli).astype(jnp.float32)
        eq_r = (iota_r == tri).astype(jnp.float32)
        v_blocks.append(fgm * (wl * eq_l + wr * eq_r))
    v_w = jnp.concatenate(v_blocks, axis=0)

    dist = dist_ref[0]
    gterm = jnp.dot(v_w, dist, preferred_element_type=jnp.float32)
    eye_d = (jax.lax.broadcasted_iota(jnp.int32, (D, D), 0)
             == jax.lax.broadcasted_iota(jnp.int32, (D, D), 1)).astype(jnp.float32)
    gath = jnp.sum(gterm * eye_d)

    dmax = jnp.max(dist, axis=1, keepdims=True)
    edist = jnp.exp(dist - dmax)
    gsel = (jax.lax.broadcasted_iota(jnp.int32, (D, 4), 0) // REG_MAX
            == jax.lax.broadcasted_iota(jnp.int32, (D, 4), 1)).astype(jnp.float32)
    gsum = jnp.dot(edist, gsel, preferred_element_type=jnp.float32)
    lse = dmax + jnp.log(gsum)
    fglse = jnp.dot(fgm, lse, preferred_element_type=jnp.float32)
    dfl_sum = jnp.sum(fglse) - gath

    zero = jnp.zeros((), jnp.float32)
    row = jnp.concatenate(
        [p.reshape(1, 1, 1) for p in
         (softplus_sum, bce_g, score_sum, box_sum, nfg, dfl_sum, zero, zero)],
        axis=2)
    out_ref[...] = row


@jax.jit
def kernel(pred_scores, pred_dist, pred_bboxes, anchors, strides,
           gt_labels, gt_bboxes, mask_gt):
    B, A, C = pred_scores.shape
    G = gt_bboxes.shape[1]
    pboxT = jnp.transpose(pred_bboxes, (0, 2, 1))
    anchT = jnp.transpose(anchors, (1, 0))
    strideT = strides.reshape(1, A)
    gtlab = gt_labels.astype(jnp.float32)[:, :, None]
    mg = mask_gt.astype(jnp.float32)[:, :, None]

    partials = pl.pallas_call(
        _loss_kernel,
        grid=(B,),
        in_specs=[
            pl.BlockSpec((1, A, C), lambda b: (b, 0, 0)),
            pl.BlockSpec((1, A, 4 * REG_MAX), lambda b: (b, 0, 0)),
            pl.BlockSpec((1, 4, A), lambda b: (b, 0, 0)),
            pl.BlockSpec((2, A), lambda b: (0, 0)),
            pl.BlockSpec((1, A), lambda b: (0, 0)),
            pl.BlockSpec((1, G, 4), lambda b: (b, 0, 0)),
            pl.BlockSpec((1, G, 1), lambda b: (b, 0, 0)),
            pl.BlockSpec((1, G, 1), lambda b: (b, 0, 0)),
        ],
        out_specs=pl.BlockSpec((1, 1, 8), lambda b: (b, 0, 0)),
        out_shape=jax.ShapeDtypeStruct((B, 1, 8), jnp.float32),
        compiler_params=pltpu.CompilerParams(
            dimension_semantics=("parallel",)),
    )(pred_scores, pred_dist, pboxT, anchT, strideT, gt_bboxes, gtlab, mg)

    partials = partials[:, 0, :]
    softplus_sum = jnp.sum(partials[:, 0])
    bce_g = jnp.sum(partials[:, 1])
    score_sum = jnp.maximum(jnp.sum(partials[:, 2]), 1.0)
    box_sum = jnp.sum(partials[:, 3])
    nfg = jnp.sum(partials[:, 4])
    dfl_sum = jnp.sum(partials[:, 5])

    loss_cls = (softplus_sum - bce_g) / score_sum
    loss_box = box_sum / nfg
    loss_dfl = dfl_sum / nfg / 4.0
    return BOX_W * loss_box + CLS_W * loss_cls + DFL_W * loss_dfl
